# Initial kernel scaffold; baseline (speedup 1.0000x reference)
import jax
import jax.numpy as jnp
from jax.experimental import pallas as pl


def kernel(x, pe_learn, mix_weight):
    raise NotImplementedError("write your pallas kernel here")


# speedup vs baseline: 3.3800x; 3.3800x over previous
import numpy as np
import jax
import jax.numpy as jnp
from jax.experimental import pallas as pl
from jax.experimental.pallas import tpu as pltpu

_D_MODEL = 2048
_BS = 256


def _sin_table(seq_len):
    position = jnp.arange(seq_len, dtype=jnp.float32)[:, None]
    div_term = jnp.exp(
        jnp.arange(0, _D_MODEL, 2, dtype=jnp.float32)
        * (-np.log(10000.0) / _D_MODEL)
    )
    ang = position * div_term
    pe = jnp.zeros((seq_len, _D_MODEL), dtype=jnp.float32)
    pe = pe.at[:, 0::2].set(jnp.sin(ang))
    pe = pe.at[:, 1::2].set(jnp.cos(ang))
    return pe


def _body(mw_ref, x_ref, sin_ref, learn_ref, o_ref):
    w = jax.nn.sigmoid(mw_ref[0, 0])
    comb = w * sin_ref[...] + (1.0 - w) * learn_ref[...]
    for b in range(x_ref.shape[0]):
        o_ref[b] = x_ref[b] + comb


def kernel(x, pe_learn, mix_weight):
    B, S, D = x.shape
    pe_sin = _sin_table(S)
    mw = jnp.asarray(mix_weight, jnp.float32).reshape(1, 1)
    return pl.pallas_call(
        _body,
        grid=(S // _BS,),
        in_specs=[
            pl.BlockSpec(memory_space=pltpu.SMEM),
            pl.BlockSpec((B, _BS, D), lambda i: (0, i, 0)),
            pl.BlockSpec((_BS, D), lambda i: (i, 0)),
            pl.BlockSpec((_BS, D), lambda i: (i, 0)),
        ],
        out_specs=pl.BlockSpec((B, _BS, D), lambda i: (0, i, 0)),
        out_shape=jax.ShapeDtypeStruct((B, S, D), x.dtype),
        compiler_params=pltpu.CompilerParams(
            dimension_semantics=("arbitrary",),
        ),
    )(mw, x, pe_sin, pe_learn[:S])
